# baseline (speedup 1.0000x reference)
import jax
import jax.numpy as jnp
from jax import lax
from jax.experimental import pallas as pl
from jax.experimental.pallas import tpu as pltpu
from jax.experimental.pallas import tpu_sc as plsc

_NC, _NS = 2, 16
_NW = _NC * _NS
_L = 16


def _gather_body(i_idx_hbm, j_idx_hbm, wi_hbm, wj_hbm, bi_hbm, bj_hbm,
                 out_i, out_j, out_bias,
                 idx_i_v, idx_j_v, rows_i_v, rows_j_v,
                 browi_v, browj_v, rowidx_i_v, rowidx_j_v, bias_v, sem):
    bpw = idx_i_v.shape[0]
    wid = lax.axis_index("s") * _NC + lax.axis_index("c")
    base = wid * bpw
    pltpu.sync_copy(i_idx_hbm.at[pl.ds(base, bpw)], idx_i_v)
    pltpu.sync_copy(j_idx_hbm.at[pl.ds(base, bpw)], idx_j_v)
    for k in range(bpw // _L):
        sl = pl.ds(k * _L, _L)
        rowidx_i_v[sl] = lax.shift_right_logical(idx_i_v[sl], 4)
        rowidx_j_v[sl] = lax.shift_right_logical(idx_j_v[sl], 4)
    c1 = pltpu.async_copy(wi_hbm.at[idx_i_v], rows_i_v, sem)
    c2 = pltpu.async_copy(wj_hbm.at[idx_j_v], rows_j_v, sem)
    c3 = pltpu.async_copy(bi_hbm.at[rowidx_i_v], browi_v, sem)
    c4 = pltpu.async_copy(bj_hbm.at[rowidx_j_v], browj_v, sem)
    c1.wait()
    c2.wait()
    c3.wait()
    c4.wait()
    for k in range(bpw // _L):
        sl = pl.ds(k * _L, _L)
        rid = lax.iota(jnp.int32, _L) + k * _L
        bi_vals = plsc.load_gather(browi_v, [rid, idx_i_v[sl] & 15])
        bj_vals = plsc.load_gather(browj_v, [rid, idx_j_v[sl] & 15])
        bias_v[sl] = bi_vals + bj_vals
    pltpu.sync_copy(rows_i_v, out_i.at[pl.ds(base, bpw)])
    pltpu.sync_copy(rows_j_v, out_j.at[pl.ds(base, bpw)])
    pltpu.sync_copy(bias_v, out_bias.at[pl.ds(base, bpw)])


def _sc_gather(i_idx, j_idx, wi, wj, bi16, bj16):
    b = i_idx.shape[0]
    d = wi.shape[1]
    bpw = b // _NW
    mesh = plsc.VectorSubcoreMesh(core_axis_name="c", subcore_axis_name="s")
    return pl.kernel(
        _gather_body,
        out_type=(
            jax.ShapeDtypeStruct((b, d), jnp.float32),
            jax.ShapeDtypeStruct((b, d), jnp.float32),
            jax.ShapeDtypeStruct((b,), jnp.float32),
        ),
        mesh=mesh,
        scratch_types=(
            pltpu.VMEM((bpw,), jnp.int32),
            pltpu.VMEM((bpw,), jnp.int32),
            pltpu.VMEM((bpw, d), jnp.float32),
            pltpu.VMEM((bpw, d), jnp.float32),
            pltpu.VMEM((bpw, _L), jnp.float32),
            pltpu.VMEM((bpw, _L), jnp.float32),
            pltpu.VMEM((bpw,), jnp.int32),
            pltpu.VMEM((bpw,), jnp.int32),
            pltpu.VMEM((bpw,), jnp.float32),
            pltpu.SemaphoreType.DMA,
        ),
        compiler_params=pltpu.CompilerParams(
            use_tc_tiling_on_sc=False, needs_layout_passes=False),
    )(i_idx, j_idx, wi, wj, bi16, bj16)


def _matmul_body(a_ref, b_ref, bias_ref, o_ref):
    o_ref[...] = lax.dot_general(
        a_ref[...], b_ref[...], (((1,), (1,)), ((), ())),
        preferred_element_type=jnp.float32,
        precision=lax.Precision.HIGHEST,
    ) + bias_ref[...]


def _tc_matmul(rows_i, rows_j, bias):
    b, d = rows_i.shape
    bm = 512
    return pl.pallas_call(
        _matmul_body,
        grid=(b // bm,),
        in_specs=[
            pl.BlockSpec((bm, d), lambda r: (r, 0)),
            pl.BlockSpec((b, d), lambda r: (0, 0)),
            pl.BlockSpec((1, b), lambda r: (0, 0)),
        ],
        out_specs=pl.BlockSpec((bm, b), lambda r: (r, 0)),
        out_shape=jax.ShapeDtypeStruct((b, b), jnp.float32),
    )(rows_i, rows_j, bias)


def kernel(i_idx, j_idx, wi, wj, bi, bj):
    vocab = wi.shape[0]
    bi16 = bi.reshape(vocab // _L, _L)
    bj16 = bj.reshape(vocab // _L, _L)
    rows_i, rows_j, bias = _sc_gather(
        i_idx.astype(jnp.int32), j_idx.astype(jnp.int32), wi, wj, bi16, bj16)
    return _tc_matmul(rows_i, rows_j, bias.reshape(1, -1))
